# Initial kernel scaffold; baseline (speedup 1.0000x reference)
import jax
import jax.numpy as jnp
from jax.experimental import pallas as pl


def kernel(x, edge_index, W1, b1, W2, b2):
    raise NotImplementedError("write your pallas kernel here")


# speedup vs baseline: 27.0611x; 27.0611x over previous
import jax
import jax.numpy as jnp
from jax import lax
from jax.experimental import pallas as pl
from jax.experimental.pallas import tpu as pltpu
from jax.experimental.pallas import tpu_sc as plsc

_N = 10000
_E = 320000
_DIN = 128
_DHID = 256
_NCLS = 16

_NC = 2
_NS = 16
_NW = _NC * _NS
_K = 80
_WIN = _E // (_NW * _K)
_NPAD = 10240
_RPT = _NPAD // _NS
_ZR = 128

def _mesh():
    return plsc.VectorSubcoreMesh(core_axis_name="c", subcore_axis_name="s",
                                  num_cores=_NC, num_subcores=_NS)


def _fill_rows(ref, n_rows, width, value):
    vec = jnp.full((16,), value, jnp.float32)

    def body(i, _):
        for k in range(width // 16):
            ref[i, pl.ds(k * 16, 16)] = vec
        return 0

    lax.fori_loop(0, n_rows, body, 0)


def _sc_aggregate(width, gather, stage_src=False):
    scratch = [
        pltpu.VMEM_SHARED((_NPAD, width), jnp.float32),
        pltpu.VMEM((_WIN, _K), jnp.int32),
        pltpu.VMEM((_K, width), jnp.float32),
        pltpu.SemaphoreType.DMA,
    ]
    if gather:
        scratch.insert(2, pltpu.VMEM((_WIN, _K), jnp.int32))
    if stage_src:
        scratch.insert(0, pltpu.VMEM_SHARED((_NPAD, width), jnp.float32))

    def body(*refs):
        if gather and stage_src:
            (rows_hbm, src_hbm, dst_hbm, out_hbm,
             src_sh, acc_sh, didx, sidx, rows, sem) = refs
        elif gather:
            (rows_hbm, src_hbm, dst_hbm, out_hbm,
             acc_sh, didx, sidx, rows, sem) = refs
        else:
            (dst_hbm, out_hbm, acc_sh, didx, rows, sem) = refs
        c = lax.axis_index("c")
        s = lax.axis_index("s")
        wid = s * _NC + c

        _fill_rows(rows, _K, width, 0.0)
        for i in range(_RPT // _K):
            pltpu.sync_copy(rows, acc_sh.at[pl.ds(s * _RPT + i * _K, _K)])
        if not gather:
            _fill_rows(rows, _K, width, 1.0)
        if stage_src:
            for i in range(_RPT // _K):
                pltpu.sync_copy(rows_hbm.at[pl.ds(s * _RPT + i * _K, _K)],
                                rows)
                pltpu.sync_copy(rows, src_sh.at[pl.ds(s * _RPT + i * _K, _K)])
        pltpu.sync_copy(dst_hbm.at[wid], didx)
        if gather:
            pltpu.sync_copy(src_hbm.at[wid], sidx)
        plsc.subcore_barrier()

        gsrc = src_sh if stage_src else (rows_hbm if gather else None)

        def win(j, _):
            if gather:
                pltpu.async_copy(gsrc.at[sidx.at[j]], rows, sem).wait()
            pltpu.sync_copy(rows, acc_sh.at[didx.at[j]], add=True)
            return 0

        lax.fori_loop(0, _WIN, win, 0)
        plsc.subcore_barrier()
        pltpu.sync_copy(acc_sh.at[pl.ds(s * _RPT, _RPT)],
                        out_hbm.at[c, pl.ds(s * _RPT, _RPT)])

    return pl.kernel(
        body,
        out_type=jax.ShapeDtypeStruct((_NC, _NPAD, width), jnp.float32),
        mesh=_mesh(),
        scratch_types=scratch,
    )


def _dinv_of(degp_ref):
    deg = degp_ref[0, :, 0:1] + degp_ref[1, :, 0:1] + 1.0
    return lax.rsqrt(deg)


_RB = 1000


def _mid_body(p_ref, xs_ref, degp_ref, w1_ref, b1_ref, w2_ref, o_ref):
    dinv = _dinv_of(degp_ref)
    g = (p_ref[0] + p_ref[1] + xs_ref[...]) * dinv
    h = jnp.dot(g, w1_ref[...], precision=lax.Precision.HIGHEST,
                preferred_element_type=jnp.float32) + b1_ref[...]
    h = jnp.maximum(h, 0.0)
    z = jnp.dot(h, w2_ref[...], precision=lax.Precision.HIGHEST,
                preferred_element_type=jnp.float32)
    o_ref[...] = z


def _out_body(p2_ref, zs_ref, degp_ref, b2_ref, lsm_ref, logits_ref):
    dinv = _dinv_of(degp_ref)
    logits = (p2_ref[0] + p2_ref[1] + zs_ref[...]) * dinv + b2_ref[...]
    m = jnp.max(logits, axis=1, keepdims=True)
    lse = jnp.log(jnp.sum(jnp.exp(logits - m), axis=1, keepdims=True)) + m
    lsm_ref[...] = logits - lse
    logits_ref[...] = logits


def _degp_spec():
    return pl.BlockSpec((_NC, _RB, _NCLS), lambda i: (0, i, 0))


def kernel(x, edge_index, W1, b1, W2, b2):
    ei = edge_index.astype(jnp.int32)
    src_r = ei[0].reshape(_NW, _WIN, _K)
    dst_r = ei[1].reshape(_NW, _WIN, _K)

    degp = _sc_aggregate(_NCLS, gather=False)(dst_r)

    dinv = lax.rsqrt(degp[0, :_N, 0] + degp[1, :_N, 0] + 1.0)
    xs = x * dinv[:, None]

    grid = _N // _RB
    p1 = _sc_aggregate(_DIN, gather=True)(xs, src_r, dst_r)

    z = pl.pallas_call(
        _mid_body,
        grid=(grid,),
        in_specs=[
            pl.BlockSpec((_NC, _RB, _DIN), lambda i: (0, i, 0)),
            pl.BlockSpec((_RB, _DIN), lambda i: (i, 0)),
            _degp_spec(),
            pl.BlockSpec((_DIN, _DHID), lambda i: (0, 0)),
            pl.BlockSpec((1, _DHID), lambda i: (0, 0)),
            pl.BlockSpec((_DHID, _NCLS), lambda i: (0, 0)),
        ],
        out_specs=pl.BlockSpec((_RB, _NCLS), lambda i: (i, 0)),
        out_shape=jax.ShapeDtypeStruct((_N, _NCLS), jnp.float32),
    )(p1, xs, degp, W1, b1.reshape(1, _DHID), W2)

    zs = z * dinv[:, None]
    zsp = jnp.pad(zs, ((0, _NPAD - _N), (0, 0)))
    p2 = _sc_aggregate(_NCLS, gather=True, stage_src=True)(zsp, src_r, dst_r)

    lsm, logits = pl.pallas_call(
        _out_body,
        grid=(grid,),
        in_specs=[
            pl.BlockSpec((_NC, _RB, _NCLS), lambda i: (0, i, 0)),
            pl.BlockSpec((_RB, _NCLS), lambda i: (i, 0)),
            _degp_spec(),
            pl.BlockSpec((1, _NCLS), lambda i: (0, 0)),
        ],
        out_specs=[
            pl.BlockSpec((_RB, _NCLS), lambda i: (i, 0)),
            pl.BlockSpec((_RB, _NCLS), lambda i: (i, 0)),
        ],
        out_shape=[
            jax.ShapeDtypeStruct((_N, _NCLS), jnp.float32),
            jax.ShapeDtypeStruct((_N, _NCLS), jnp.float32),
        ],
    )(p2, zs, degp, b2.reshape(1, _NCLS))

    return (lsm, logits)


# speedup vs baseline: 30.4323x; 1.1246x over previous
import jax
import jax.numpy as jnp
from jax import lax
from jax.experimental import pallas as pl
from jax.experimental.pallas import tpu as pltpu
from jax.experimental.pallas import tpu_sc as plsc

_N = 10000
_E = 320000
_DIN = 128
_DHID = 256
_NCLS = 16

_NC = 2
_NS = 16
_NW = _NC * _NS
_K = 128
_WIN = 79
_EP = _NW * _WIN * _K
_NPAD = 10240
_RPT = _NPAD // _NS

def _mesh():
    return plsc.VectorSubcoreMesh(core_axis_name="c", subcore_axis_name="s",
                                  num_cores=_NC, num_subcores=_NS)


def _fill_rows(ref, n_rows, width, value):
    vec = jnp.full((16,), value, jnp.float32)

    def body(i, _):
        for k in range(width // 16):
            ref[i, pl.ds(k * 16, 16)] = vec
        return 0

    lax.fori_loop(0, n_rows, body, 0)


def _sc_aggregate(width, gather, stage_src=False, n_buf=1):
    scratch = [
        pltpu.VMEM_SHARED((_NPAD, width), jnp.float32),
        pltpu.VMEM((_WIN, _K), jnp.int32),
        pltpu.VMEM((_K, width), jnp.float32),
        pltpu.SemaphoreType.DMA,
    ]
    if gather:
        scratch.insert(2, pltpu.VMEM((_WIN, _K), jnp.int32))
        for _ in range(n_buf - 1):
            scratch.append(pltpu.VMEM((_K, width), jnp.float32))
            scratch.append(pltpu.SemaphoreType.DMA)
    if stage_src:
        scratch.insert(0, pltpu.VMEM_SHARED((_NPAD, width), jnp.float32))

    def body(*refs):
        if gather and stage_src:
            (rows_hbm, src_hbm, dst_hbm, out_hbm,
             src_sh, acc_sh, didx, sidx, rows, sem, *extra) = refs
        elif gather:
            (rows_hbm, src_hbm, dst_hbm, out_hbm,
             acc_sh, didx, sidx, rows, sem, *extra) = refs
        else:
            (dst_hbm, out_hbm, acc_sh, didx, rows, sem) = refs
        if gather:
            bufs = [rows] + list(extra[0::2])
            sems = [sem] + list(extra[1::2])
        c = lax.axis_index("c")
        s = lax.axis_index("s")
        wid = s * _NC + c

        _fill_rows(rows, _K, width, 0.0)
        for i in range(_RPT // _K):
            pltpu.sync_copy(rows, acc_sh.at[pl.ds(s * _RPT + i * _K, _K)])
        if not gather:
            _fill_rows(rows, _K, width, 1.0)
        if stage_src:
            for i in range(_RPT // _K):
                pltpu.sync_copy(rows_hbm.at[pl.ds(s * _RPT + i * _K, _K)],
                                rows)
                pltpu.sync_copy(rows, src_sh.at[pl.ds(s * _RPT + i * _K, _K)])
        pltpu.sync_copy(dst_hbm.at[wid], didx)
        if gather:
            pltpu.sync_copy(src_hbm.at[wid], sidx)
        plsc.subcore_barrier()

        if gather:
            gsrc = src_sh if stage_src else rows_hbm

            def sl(j):
                return sidx.at[j]

            def group(g, _):
                j0 = g * n_buf
                descs = [
                    pltpu.async_copy(gsrc.at[sl(j0 + t)], bufs[t], sems[t])
                    for t in range(n_buf)
                ]
                for t in range(n_buf):
                    descs[t].wait()
                    pltpu.sync_copy(bufs[t], acc_sh.at[didx.at[j0 + t]],
                                    add=True)
                return 0

            lax.fori_loop(0, _WIN // n_buf, group, 0)
            for j in range(_WIN - _WIN % n_buf, _WIN):
                pltpu.async_copy(gsrc.at[sl(j)], rows, sem).wait()
                pltpu.sync_copy(rows, acc_sh.at[didx.at[j]], add=True)
        else:
            def win(j, _):
                pltpu.sync_copy(rows, acc_sh.at[didx.at[j]], add=True)
                return 0

            lax.fori_loop(0, _WIN, win, 0)
        plsc.subcore_barrier()
        pltpu.sync_copy(acc_sh.at[pl.ds(s * _RPT, _RPT)],
                        out_hbm.at[c, pl.ds(s * _RPT, _RPT)])

    return pl.kernel(
        body,
        out_type=jax.ShapeDtypeStruct((_NC, _NPAD, width), jnp.float32),
        mesh=_mesh(),
        scratch_types=scratch,
    )


def _dinv_of(degp_ref):
    deg = degp_ref[0, :, 0:1] + degp_ref[1, :, 0:1] + 1.0
    return lax.rsqrt(deg)


_RB = 1000


def _mid_body(p_ref, xs_ref, degp_ref, w1_ref, b1_ref, w2_ref, o_ref):
    dinv = _dinv_of(degp_ref)
    g = (p_ref[0] + p_ref[1] + xs_ref[...]) * dinv
    h = jnp.dot(g, w1_ref[...], precision=lax.Precision.HIGHEST,
                preferred_element_type=jnp.float32) + b1_ref[...]
    h = jnp.maximum(h, 0.0)
    z = jnp.dot(h, w2_ref[...], precision=lax.Precision.HIGHEST,
                preferred_element_type=jnp.float32)
    o_ref[...] = z


def _out_body(p2_ref, zs_ref, degp_ref, b2_ref, lsm_ref, logits_ref):
    dinv = _dinv_of(degp_ref)
    logits = (p2_ref[0] + p2_ref[1] + zs_ref[...]) * dinv + b2_ref[...]
    m = jnp.max(logits, axis=1, keepdims=True)
    lse = jnp.log(jnp.sum(jnp.exp(logits - m), axis=1, keepdims=True)) + m
    lsm_ref[...] = logits - lse
    logits_ref[...] = logits


def _degp_spec():
    return pl.BlockSpec((_NC, _RB, _NCLS), lambda i: (0, i, 0))


def kernel(x, edge_index, W1, b1, W2, b2):
    ei = edge_index.astype(jnp.int32)
    pad_n = _EP - _E
    apad = jnp.arange(pad_n, dtype=jnp.int32)
    src_pad = (apad * 97) % _N
    dst_pad = _N + apad % (_NPAD - _N)
    src_r = jnp.concatenate([ei[0], src_pad]).reshape(_NW, _WIN, _K)
    dst_r = jnp.concatenate([ei[1], dst_pad]).reshape(_NW, _WIN, _K)

    degp = _sc_aggregate(_NCLS, gather=False)(dst_r)

    dinv = lax.rsqrt(degp[0, :_N, 0] + degp[1, :_N, 0] + 1.0)
    xs = x * dinv[:, None]

    grid = _N // _RB
    p1 = _sc_aggregate(_DIN, gather=True)(xs, src_r, dst_r)

    z = pl.pallas_call(
        _mid_body,
        grid=(grid,),
        in_specs=[
            pl.BlockSpec((_NC, _RB, _DIN), lambda i: (0, i, 0)),
            pl.BlockSpec((_RB, _DIN), lambda i: (i, 0)),
            _degp_spec(),
            pl.BlockSpec((_DIN, _DHID), lambda i: (0, 0)),
            pl.BlockSpec((1, _DHID), lambda i: (0, 0)),
            pl.BlockSpec((_DHID, _NCLS), lambda i: (0, 0)),
        ],
        out_specs=pl.BlockSpec((_RB, _NCLS), lambda i: (i, 0)),
        out_shape=jax.ShapeDtypeStruct((_N, _NCLS), jnp.float32),
    )(p1, xs, degp, W1, b1.reshape(1, _DHID), W2)

    zs = z * dinv[:, None]
    zsp = jnp.pad(zs, ((0, _NPAD - _N), (0, 0)))
    p2 = _sc_aggregate(_NCLS, gather=True, stage_src=True)(zsp, src_r, dst_r)

    lsm, logits = pl.pallas_call(
        _out_body,
        grid=(grid,),
        in_specs=[
            pl.BlockSpec((_NC, _RB, _NCLS), lambda i: (0, i, 0)),
            pl.BlockSpec((_RB, _NCLS), lambda i: (i, 0)),
            _degp_spec(),
            pl.BlockSpec((1, _NCLS), lambda i: (0, 0)),
        ],
        out_specs=[
            pl.BlockSpec((_RB, _NCLS), lambda i: (i, 0)),
            pl.BlockSpec((_RB, _NCLS), lambda i: (i, 0)),
        ],
        out_shape=[
            jax.ShapeDtypeStruct((_N, _NCLS), jnp.float32),
            jax.ShapeDtypeStruct((_N, _NCLS), jnp.float32),
        ],
    )(p2, zs, degp, b2.reshape(1, _NCLS))

    return (lsm, logits)


# speedup vs baseline: 31.1592x; 1.0239x over previous
import jax
import jax.numpy as jnp
from jax import lax
from jax.experimental import pallas as pl
from jax.experimental.pallas import tpu as pltpu
from jax.experimental.pallas import tpu_sc as plsc

_N = 10000
_E = 320000
_DIN = 128
_DHID = 256
_NCLS = 16

_NC = 2
_NS = 16
_NW = _NC * _NS
_K = 128
_WIN = 79
_EP = _NW * _WIN * _K
_NPAD = 10240
_RPT = _NPAD // _NS

def _mesh():
    return plsc.VectorSubcoreMesh(core_axis_name="c", subcore_axis_name="s",
                                  num_cores=_NC, num_subcores=_NS)


def _fill_rows(ref, n_rows, width, value):
    vec = jnp.full((16,), value, jnp.float32)

    def body(i, _):
        for k in range(width // 16):
            ref[i, pl.ds(k * 16, 16)] = vec
        return 0

    lax.fori_loop(0, n_rows, body, 0)


def _sc_aggregate(width, gather, stage_src=False, n_buf=1):
    scratch = [
        pltpu.VMEM_SHARED((_NPAD, width), jnp.float32),
        pltpu.VMEM((_WIN, _K), jnp.int32),
        pltpu.VMEM((_K, width), jnp.float32),
        pltpu.SemaphoreType.DMA,
    ]
    if gather:
        scratch.insert(2, pltpu.VMEM((_WIN, _K), jnp.int32))
        for _ in range(n_buf - 1):
            scratch.append(pltpu.VMEM((_K, width), jnp.float32))
            scratch.append(pltpu.SemaphoreType.DMA)
    if stage_src:
        scratch.insert(0, pltpu.VMEM_SHARED((_NPAD, width), jnp.float32))

    def body(*refs):
        if gather and stage_src:
            (rows_hbm, src_hbm, dst_hbm, out_hbm,
             src_sh, acc_sh, didx, sidx, rows, sem, *extra) = refs
        elif gather:
            (rows_hbm, src_hbm, dst_hbm, out_hbm,
             acc_sh, didx, sidx, rows, sem, *extra) = refs
        else:
            (dst_hbm, out_hbm, acc_sh, didx, rows, sem) = refs
        if gather:
            bufs = [rows] + list(extra[0::2])
            sems = [sem] + list(extra[1::2])
        c = lax.axis_index("c")
        s = lax.axis_index("s")
        wid = s * _NC + c

        _fill_rows(rows, _K, width, 0.0)
        for i in range(_RPT // _K):
            pltpu.sync_copy(rows, acc_sh.at[pl.ds(s * _RPT + i * _K, _K)])
        if not gather:
            _fill_rows(rows, _K, width, 1.0)
        if stage_src:
            for i in range(_RPT // _K):
                pltpu.sync_copy(rows_hbm.at[pl.ds(s * _RPT + i * _K, _K)],
                                rows)
                pltpu.sync_copy(rows, src_sh.at[pl.ds(s * _RPT + i * _K, _K)])
        pltpu.sync_copy(dst_hbm.at[wid], didx)
        if gather:
            pltpu.sync_copy(src_hbm.at[wid], sidx)
        plsc.subcore_barrier()

        if gather:
            gsrc = src_sh if stage_src else rows_hbm

            def sl(j):
                return sidx.at[j]

            def group(g, _):
                j0 = g * n_buf
                descs = [
                    pltpu.async_copy(gsrc.at[sl(j0 + t)], bufs[t], sems[t])
                    for t in range(n_buf)
                ]
                for t in range(n_buf):
                    descs[t].wait()
                    pltpu.sync_copy(bufs[t], acc_sh.at[didx.at[j0 + t]],
                                    add=True)
                return 0

            lax.fori_loop(0, _WIN // n_buf, group, 0)
            for j in range(_WIN - _WIN % n_buf, _WIN):
                pltpu.async_copy(gsrc.at[sl(j)], rows, sem).wait()
                pltpu.sync_copy(rows, acc_sh.at[didx.at[j]], add=True)
        else:
            def win(j, _):
                pltpu.sync_copy(rows, acc_sh.at[didx.at[j]], add=True)
                return 0

            lax.fori_loop(0, _WIN, win, 0)
        plsc.subcore_barrier()
        pltpu.sync_copy(acc_sh.at[pl.ds(s * _RPT, _RPT)],
                        out_hbm.at[c, pl.ds(s * _RPT, _RPT)])

    return pl.kernel(
        body,
        out_type=jax.ShapeDtypeStruct((_NC, _NPAD, width), jnp.float32),
        mesh=_mesh(),
        scratch_types=scratch,
    )


def _dinv_of(degp_ref):
    deg = degp_ref[0, :, 0:1] + degp_ref[1, :, 0:1] + 1.0
    return lax.rsqrt(deg)


_RB = 1000


def _mid_body(p_ref, xs_ref, degp_ref, w1_ref, b1_ref, w2_ref, o_ref):
    dinv = _dinv_of(degp_ref)
    g = (p_ref[0] + p_ref[1] + xs_ref[...]) * dinv
    h = jnp.dot(g, w1_ref[...], precision=lax.Precision.HIGHEST,
                preferred_element_type=jnp.float32) + b1_ref[...]
    h = jnp.maximum(h, 0.0)
    z = jnp.dot(h, w2_ref[...], precision=lax.Precision.HIGHEST,
                preferred_element_type=jnp.float32)
    o_ref[...] = z


def _out_body(p2_ref, zs_ref, degp_ref, b2_ref, lsm_ref, logits_ref):
    dinv = _dinv_of(degp_ref)
    logits = (p2_ref[0] + p2_ref[1] + zs_ref[...]) * dinv + b2_ref[...]
    m = jnp.max(logits, axis=1, keepdims=True)
    lse = jnp.log(jnp.sum(jnp.exp(logits - m), axis=1, keepdims=True)) + m
    lsm_ref[...] = logits - lse
    logits_ref[...] = logits


def _degp_spec():
    return pl.BlockSpec((_NC, _RB, _NCLS), lambda i: (0, i, 0))


def kernel(x, edge_index, W1, b1, W2, b2):
    ei = edge_index.astype(jnp.int32)
    pad_n = _EP - _E
    apad = jnp.arange(pad_n, dtype=jnp.int32)
    src_pad = (apad * 97) % _N
    dst_pad = _N + apad % (_NPAD - _N)
    src_r = jnp.concatenate([ei[0], src_pad]).reshape(_NW, _WIN, _K)
    dst_r = jnp.concatenate([ei[1], dst_pad]).reshape(_NW, _WIN, _K)

    degp = _sc_aggregate(_NCLS, gather=False)(dst_r)

    dinv = lax.rsqrt(degp[0, :_N, 0] + degp[1, :_N, 0] + 1.0)
    xs = x * dinv[:, None]

    grid = _N // _RB
    p1 = _sc_aggregate(_DIN, gather=True)(xs, src_r, dst_r)

    z = pl.pallas_call(
        _mid_body,
        grid=(grid,),
        in_specs=[
            pl.BlockSpec((_NC, _RB, _DIN), lambda i: (0, i, 0)),
            pl.BlockSpec((_RB, _DIN), lambda i: (i, 0)),
            _degp_spec(),
            pl.BlockSpec((_DIN, _DHID), lambda i: (0, 0)),
            pl.BlockSpec((1, _DHID), lambda i: (0, 0)),
            pl.BlockSpec((_DHID, _NCLS), lambda i: (0, 0)),
        ],
        out_specs=pl.BlockSpec((_RB, _NCLS), lambda i: (i, 0)),
        out_shape=jax.ShapeDtypeStruct((_N, _NCLS), jnp.float32),
    )(p1, xs, degp, W1, b1.reshape(1, _DHID), W2)

    zs = z * dinv[:, None]
    zsp = jnp.pad(zs, ((0, _NPAD - _N), (0, 0)))
    p2 = _sc_aggregate(_NCLS, gather=True, stage_src=True,
                       n_buf=4)(zsp, src_r, dst_r)

    lsm, logits = pl.pallas_call(
        _out_body,
        grid=(grid,),
        in_specs=[
            pl.BlockSpec((_NC, _RB, _NCLS), lambda i: (0, i, 0)),
            pl.BlockSpec((_RB, _NCLS), lambda i: (i, 0)),
            _degp_spec(),
            pl.BlockSpec((1, _NCLS), lambda i: (0, 0)),
        ],
        out_specs=[
            pl.BlockSpec((_RB, _NCLS), lambda i: (i, 0)),
            pl.BlockSpec((_RB, _NCLS), lambda i: (i, 0)),
        ],
        out_shape=[
            jax.ShapeDtypeStruct((_N, _NCLS), jnp.float32),
            jax.ShapeDtypeStruct((_N, _NCLS), jnp.float32),
        ],
    )(p2, zs, degp, b2.reshape(1, _NCLS))

    return (lsm, logits)
